# baseline (speedup 1.0000x reference)
import functools

import jax
import jax.numpy as jnp
import numpy as np
from jax import lax
from jax.experimental import pallas as pl
from jax.experimental.pallas import tpu as pltpu
from jax.experimental.pallas import tpu_sc as plsc

FEAT = 100000
D = 256
BAG = 50
PB = 112
NBAGS = 5 * 4096
NST = NBAGS // 2
NW = 32
BPW = NBAGS // NW
SPW = NST // NW
G = 8
SG = G // 2
RDEPTH = 4
PAIRS = SPW // (2 * SG)
LANES = 16
NCH = D // LANES

CLIP = 127.0 / 128.0
CLIPC = 1.0


def _sc_bag_sums(idx_all, table):
    mesh = plsc.VectorSubcoreMesh(core_axis_name="c", subcore_axis_name="s")

    @functools.partial(
        pl.kernel,
        mesh=mesh,
        out_type=jax.ShapeDtypeStruct((NBAGS, D), jnp.float32),
        scratch_types=[
            pltpu.VMEM((2, SG, PB), jnp.int32),
            pltpu.VMEM((RDEPTH, PB, D // 2), jnp.uint32),
            pltpu.VMEM((2, G, D), jnp.float32),
            pltpu.SemaphoreType.DMA,
            pltpu.SemaphoreType.DMA,
            pltpu.SemaphoreType.DMA,
            pltpu.SemaphoreType.DMA,
            pltpu.SemaphoreType.DMA,
            pltpu.SemaphoreType.DMA,
        ],
    )
    def body(idx_hbm, table_hbm, out_hbm, idx_c, rbuf, obuf,
             gs0, gs1, gs2, gs3, os0, os1):
        gsem = (gs0, gs1, gs2, gs3)
        osem = (os0, os1)
        wid = lax.axis_index("s") * 2 + lax.axis_index("c")
        base = wid * BPW
        sbase = wid * SPW
        nsec = SPW // SG
        pltpu.sync_copy(idx_hbm.at[pl.ds(sbase, SG)], idx_c.at[0])
        for b0 in range(RDEPTH - 1):
            pltpu.async_copy(
                table_hbm.at[idx_c.at[0].at[b0]], rbuf.at[b0], gsem[b0]
            )

        def wait_gather(slot, h, j):
            pltpu.make_async_copy(
                table_hbm.at[idx_c.at[h].at[j]], rbuf.at[slot], gsem[slot]
            ).wait()

        def reduce_bag(slot, h, jbag, off):
            rb = rbuf.at[slot]
            mask = jnp.full((LANES,), 0xFFFF0000, jnp.uint32)

            def rbody(r, accs):
                accs = list(accs)
                for c2 in range(NCH // 2):
                    u = rb[r + off, pl.ds(c2 * LANES, LANES)]
                    lo = lax.bitcast_convert_type(u << 16, jnp.float32)
                    hi = lax.bitcast_convert_type(u & mask, jnp.float32)
                    accs[2 * c2] = accs[2 * c2] + lo
                    accs[2 * c2 + 1] = accs[2 * c2 + 1] + hi
                return tuple(accs)

            init = tuple(jnp.zeros((LANES,), jnp.float32) for _ in range(NCH))
            accs = lax.fori_loop(0, BAG, rbody, init, unroll=5)
            for c2 in range(NCH // 2):
                obuf[h, jbag, pl.ds(c2 * 2 * LANES, LANES)] = accs[2 * c2]
                obuf[h, jbag, pl.ds(c2 * 2 * LANES + LANES, LANES)] = accs[2 * c2 + 1]

        def pair_body(pi, carry):
            for h in range(2):
                grp = pi * 2 + h

                @pl.when(grp + 1 < nsec)
                def _():
                    pltpu.sync_copy(
                        idx_hbm.at[pl.ds(sbase + (grp + 1) * SG, SG)],
                        idx_c.at[1 - h],
                    )

                @pl.when(pi >= 1)
                def _():
                    pltpu.make_async_copy(
                        obuf.at[h], out_hbm.at[pl.ds(base, G)], osem[h]
                    ).wait()

                for j in range(SG):
                    s = grp * SG + j
                    cur = j % RDEPTH
                    nxt = (j + RDEPTH - 1) % RDEPTH
                    la = j + RDEPTH - 1
                    if la < SG:
                        idxref = idx_c.at[h].at[la]
                    else:
                        idxref = idx_c.at[1 - h].at[la - SG]

                    @pl.when(s + RDEPTH - 1 < SPW)
                    def _():
                        pltpu.async_copy(
                            table_hbm.at[idxref], rbuf.at[nxt], gsem[nxt]
                        )

                    wait_gather(cur, h, j)
                    reduce_bag(cur, h, 2 * j, 0)
                    reduce_bag(cur, h, 2 * j + 1, BAG)
                pltpu.async_copy(
                    obuf.at[h], out_hbm.at[pl.ds(base + grp * G, G)], osem[h]
                )
            return carry

        lax.fori_loop(0, PAIRS, pair_body, 0)
        for h in range(2):
            pltpu.make_async_copy(
                obuf.at[h], out_hbm.at[pl.ds(base, G)], osem[h]
            ).wait()

    return body(idx_all, table)


def _mlp_head(sums5, wct, bc, w2t, b2, w3t, b3, w4t, b4):
    batch = sums5.shape[1]
    blk = 512
    grid = (batch // blk,)

    def body(s_ref, wc_ref, bc_ref, w2_ref, b2_ref, w3_ref, b3_ref, w4_ref,
             b4_ref, o_ref):
        xs = []
        for g in range(4):
            xs.append(jnp.clip(s_ref[g], -CLIP, CLIP))
        cond = jnp.clip(s_ref[4], -CLIPC, CLIPC) + xs[0] + xs[1] + xs[2] + xs[3]
        cond = jnp.dot(cond, wc_ref[...], preferred_element_type=jnp.float32)
        cond = jnp.clip(cond + bc_ref[...], -CLIP, CLIP)
        for g in range(4):
            h = jnp.dot(xs[g], w2_ref[...], preferred_element_type=jnp.float32)
            h = jnp.clip(h + b2_ref[...], -CLIP, CLIP)
            h = h + cond
            h = jnp.dot(h, w3_ref[...], preferred_element_type=jnp.float32)
            h = jnp.clip(h + b3_ref[...], -CLIP, CLIP)
            o = jnp.dot(h, w4_ref[...], preferred_element_type=jnp.float32)
            o_ref[g] = o + b4_ref[...]

    return pl.pallas_call(
        body,
        grid=grid,
        in_specs=[
            pl.BlockSpec((5, blk, D), lambda i: (0, i, 0)),
            pl.BlockSpec((D, 32), lambda i: (0, 0)),
            pl.BlockSpec((1, 32), lambda i: (0, 0)),
            pl.BlockSpec((D, 32), lambda i: (0, 0)),
            pl.BlockSpec((1, 32), lambda i: (0, 0)),
            pl.BlockSpec((32, 32), lambda i: (0, 0)),
            pl.BlockSpec((1, 32), lambda i: (0, 0)),
            pl.BlockSpec((32, 5), lambda i: (0, 0)),
            pl.BlockSpec((1, 5), lambda i: (0, 0)),
        ],
        out_specs=pl.BlockSpec((4, blk, 5), lambda i: (0, i, 0)),
        out_shape=jax.ShapeDtypeStruct((4, batch, 5), jnp.float32),
    )(sums5, wct, bc, w2t, b2, w3t, b3, w4t, b4)


def kernel(x, condition, table, Wc, bc, W2, b2, W3, b3, W4, b4):
    batch = x.shape[0]
    xm = jnp.where(x == -100, FEAT, x).astype(jnp.int32)
    xg = jnp.transpose(xm, (1, 0, 2))
    nst = 5 * batch // 2
    idx_all = jnp.concatenate(
        [xg, condition.astype(jnp.int32)[None]], axis=0
    ).reshape(nst, 2 * BAG)
    npad = PB - 2 * BAG
    p0 = jax.lax.broadcasted_iota(jnp.int32, (nst, npad), 0)
    p1 = jax.lax.broadcasted_iota(jnp.int32, (nst, npad), 1)
    pad = ((p0 * npad + p1) * 9973) % FEAT
    idx_all = jnp.concatenate([idx_all, pad], axis=1)
    def _rne_hi16(u):
        return (u + 0x7FFF + ((u >> 16) & 1)) >> 16

    ua = jax.lax.bitcast_convert_type(table[:, : D // 2], jnp.uint32)
    ub = jax.lax.bitcast_convert_type(table[:, D // 2:], jnp.uint32)
    tb32 = _rne_hi16(ua) | (_rne_hi16(ub) << 16)
    sums = _sc_bag_sums(idx_all, tb32)
    sums5 = sums.reshape(5, batch, D)
    base = np.arange(D // 2).reshape(-1, LANES)
    operm = np.concatenate([base, base + D // 2], axis=1).reshape(D)
    out = _mlp_head(
        sums5, Wc.T[operm], bc.reshape(1, -1), W2.T[operm], b2.reshape(1, -1),
        W3.T, b3.reshape(1, -1), W4.T, b4.reshape(1, -1),
    )
    return jnp.transpose(out, (1, 0, 2))


# speedup vs baseline: 1.0167x; 1.0167x over previous
import functools

import jax
import jax.numpy as jnp
import numpy as np
from jax import lax
from jax.experimental import pallas as pl
from jax.experimental.pallas import tpu as pltpu
from jax.experimental.pallas import tpu_sc as plsc

FEAT = 100000
D = 256
BAG = 50
BAGP = 56
NBAGS = 5 * 4096
NW = 32
BPW = NBAGS // NW
G = 8
RDEPTH = 4
PAIRS = BPW // (2 * G)
LANES = 16
NCH = D // LANES

CLIP = 127.0 / 128.0
CLIPC = 1.0


def _sc_bag_sums(idx_all, table):
    mesh = plsc.VectorSubcoreMesh(core_axis_name="c", subcore_axis_name="s")

    @functools.partial(
        pl.kernel,
        mesh=mesh,
        out_type=jax.ShapeDtypeStruct((NBAGS, D), jnp.float32),
        scratch_types=[
            pltpu.VMEM((2, G, BAGP), jnp.int32),
            pltpu.VMEM((RDEPTH, BAGP, D // 2), jnp.uint32),
            pltpu.VMEM((2, G, D), jnp.float32),
            pltpu.SemaphoreType.DMA,
            pltpu.SemaphoreType.DMA,
            pltpu.SemaphoreType.DMA,
            pltpu.SemaphoreType.DMA,
            pltpu.SemaphoreType.DMA,
            pltpu.SemaphoreType.DMA,
        ],
    )
    def body(idx_hbm, table_hbm, out_hbm, idx_c, rbuf, obuf,
             gs0, gs1, gs2, gs3, os0, os1):
        gsem = (gs0, gs1, gs2, gs3)
        osem = (os0, os1)
        wid = lax.axis_index("s") * 2 + lax.axis_index("c")
        base = wid * BPW
        nsec = BPW // G
        pltpu.sync_copy(idx_hbm.at[pl.ds(base, G)], idx_c.at[0])
        for b0 in range(RDEPTH - 1):
            pltpu.async_copy(
                table_hbm.at[idx_c.at[0].at[b0]], rbuf.at[b0], gsem[b0]
            )

        def wait_gather(slot, h, j):
            pltpu.make_async_copy(
                table_hbm.at[idx_c.at[h].at[j]], rbuf.at[slot], gsem[slot]
            ).wait()

        def reduce_bag(slot, h, j):
            rb = rbuf.at[slot]
            mask = jnp.full((LANES,), 0xFFFF0000, jnp.uint32)

            def rbody(r, accs):
                accs = list(accs)
                for c2 in range(NCH // 2):
                    u = rb[r, pl.ds(c2 * LANES, LANES)]
                    lo = lax.bitcast_convert_type(u << 16, jnp.float32)
                    hi = lax.bitcast_convert_type(u & mask, jnp.float32)
                    accs[2 * c2] = accs[2 * c2] + lo
                    accs[2 * c2 + 1] = accs[2 * c2 + 1] + hi
                return tuple(accs)

            init = tuple(jnp.zeros((LANES,), jnp.float32) for _ in range(NCH))
            accs = lax.fori_loop(0, BAG, rbody, init, unroll=5)
            for c2 in range(NCH // 2):
                obuf[h, j, pl.ds(c2 * 2 * LANES, LANES)] = accs[2 * c2]
                obuf[h, j, pl.ds(c2 * 2 * LANES + LANES, LANES)] = accs[2 * c2 + 1]

        def pair_body(pi, carry):
            for h in range(2):
                grp = pi * 2 + h

                @pl.when(grp + 1 < nsec)
                def _():
                    pltpu.sync_copy(
                        idx_hbm.at[pl.ds(base + (grp + 1) * G, G)],
                        idx_c.at[1 - h],
                    )

                @pl.when(pi >= 1)
                def _():
                    pltpu.make_async_copy(
                        obuf.at[h], out_hbm.at[pl.ds(base, G)], osem[h]
                    ).wait()

                for j in range(G):
                    b = grp * G + j
                    cur = j % RDEPTH
                    nxt = (j + RDEPTH - 1) % RDEPTH
                    la = j + RDEPTH - 1
                    if la < G:
                        idxref = idx_c.at[h].at[la]
                    else:
                        idxref = idx_c.at[1 - h].at[la - G]

                    @pl.when(b + RDEPTH - 1 < BPW)
                    def _():
                        pltpu.async_copy(
                            table_hbm.at[idxref], rbuf.at[nxt], gsem[nxt]
                        )

                    wait_gather(cur, h, j)
                    reduce_bag(cur, h, j)
                pltpu.async_copy(
                    obuf.at[h], out_hbm.at[pl.ds(base + grp * G, G)], osem[h]
                )
            return carry

        lax.fori_loop(0, PAIRS, pair_body, 0)
        for h in range(2):
            pltpu.make_async_copy(
                obuf.at[h], out_hbm.at[pl.ds(base, G)], osem[h]
            ).wait()

    return body(idx_all, table)


def _mlp_head(sums5, wct, bc, w2t, b2, w3t, b3, w4t, b4):
    batch = sums5.shape[1]
    blk = 512
    grid = (batch // blk,)

    def body(s_ref, wc_ref, bc_ref, w2_ref, b2_ref, w3_ref, b3_ref, w4_ref,
             b4_ref, o_ref):
        xs = []
        for g in range(4):
            xs.append(jnp.clip(s_ref[g], -CLIP, CLIP))
        cond = jnp.clip(s_ref[4], -CLIPC, CLIPC) + xs[0] + xs[1] + xs[2] + xs[3]
        cond = jnp.dot(cond, wc_ref[...], preferred_element_type=jnp.float32)
        cond = jnp.clip(cond + bc_ref[...], -CLIP, CLIP)
        for g in range(4):
            h = jnp.dot(xs[g], w2_ref[...], preferred_element_type=jnp.float32)
            h = jnp.clip(h + b2_ref[...], -CLIP, CLIP)
            h = h + cond
            h = jnp.dot(h, w3_ref[...], preferred_element_type=jnp.float32)
            h = jnp.clip(h + b3_ref[...], -CLIP, CLIP)
            o = jnp.dot(h, w4_ref[...], preferred_element_type=jnp.float32)
            o_ref[g] = o + b4_ref[...]

    return pl.pallas_call(
        body,
        grid=grid,
        in_specs=[
            pl.BlockSpec((5, blk, D), lambda i: (0, i, 0)),
            pl.BlockSpec((D, 32), lambda i: (0, 0)),
            pl.BlockSpec((1, 32), lambda i: (0, 0)),
            pl.BlockSpec((D, 32), lambda i: (0, 0)),
            pl.BlockSpec((1, 32), lambda i: (0, 0)),
            pl.BlockSpec((32, 32), lambda i: (0, 0)),
            pl.BlockSpec((1, 32), lambda i: (0, 0)),
            pl.BlockSpec((32, 5), lambda i: (0, 0)),
            pl.BlockSpec((1, 5), lambda i: (0, 0)),
        ],
        out_specs=pl.BlockSpec((4, blk, 5), lambda i: (0, i, 0)),
        out_shape=jax.ShapeDtypeStruct((4, batch, 5), jnp.float32),
    )(sums5, wct, bc, w2t, b2, w3t, b3, w4t, b4)


def kernel(x, condition, table, Wc, bc, W2, b2, W3, b3, W4, b4):
    batch = x.shape[0]
    xm = jnp.where(x == -100, FEAT, x).astype(jnp.int32)
    xg = jnp.transpose(xm, (1, 0, 2))
    idx_all = jnp.concatenate(
        [xg, condition.astype(jnp.int32)[None]], axis=0
    ).reshape(5 * batch, BAG)
    npad = BAGP - BAG
    p0 = jax.lax.broadcasted_iota(jnp.int32, (5 * batch, npad), 0)
    p1 = jax.lax.broadcasted_iota(jnp.int32, (5 * batch, npad), 1)
    pad = ((p0 * npad + p1) * 9973) % FEAT
    idx_all = jnp.concatenate([idx_all, pad], axis=1)
    def _rne_hi16(u):
        return (u + 0x7FFF + ((u >> 16) & 1)) >> 16

    ua = jax.lax.bitcast_convert_type(table[:, : D // 2], jnp.uint32)
    ub = jax.lax.bitcast_convert_type(table[:, D // 2:], jnp.uint32)
    tb32 = _rne_hi16(ua) | (_rne_hi16(ub) << 16)
    sums = _sc_bag_sums(idx_all, tb32)
    sums5 = sums.reshape(5, batch, D)
    base = np.arange(D // 2).reshape(-1, LANES)
    operm = np.concatenate([base, base + D // 2], axis=1).reshape(D)
    out = _mlp_head(
        sums5, Wc.T[operm], bc.reshape(1, -1), W2.T[operm], b2.reshape(1, -1),
        W3.T, b3.reshape(1, -1), W4.T, b4.reshape(1, -1),
    )
    return jnp.transpose(out, (1, 0, 2))
